# baseline (speedup 1.0000x reference)
import numpy as np
import jax
import jax.numpy as jnp
from jax import lax
from jax.experimental import pallas as pl
from jax.experimental.pallas import tpu as pltpu
from jax.experimental.pallas import tpu_sc as plsc

VOCAB_N = 1000000
D = 64
BATCH = 4096
SEQ = 200
MAXLEN = 4096

NW = 32
TOTAL = BATCH * SEQ
RPW = TOTAL // NW
C = 128
NCHUNK = RPW // C


def _make_pe2():
    position = np.arange(MAXLEN, dtype=np.float32)[:, None]
    div_term = np.exp(
        np.arange(0, D, 2, dtype=np.float32) * (-np.log(10000.0) / D))
    pe = np.zeros((MAXLEN, D), dtype=np.float32)
    pe[:, 0::2] = np.sin(position * div_term)
    pe[:, 1::2] = np.cos(position * div_term)
    pe = pe[:SEQ]
    return np.concatenate([pe, pe], axis=0)


_PE2 = jnp.asarray(_make_pe2())
_IDENT = jnp.arange(C, dtype=jnp.int32)


def _sc_embed(x3, table, pe2, ident):
    mesh = plsc.VectorSubcoreMesh(core_axis_name="c", subcore_axis_name="s")

    @pl.kernel(
        out_type=jax.ShapeDtypeStruct((TOTAL, D), jnp.float32),
        mesh=mesh,
        compiler_params=pltpu.CompilerParams(use_tc_tiling_on_sc=False),
        scratch_types=[
            pltpu.VMEM((NCHUNK, C), jnp.int32),
            pltpu.VMEM((2 * SEQ, D), jnp.float32),
            pltpu.VMEM((C,), jnp.int32),
            pltpu.VMEM((C, D), jnp.float32),
            pltpu.VMEM_SHARED((16, C, D), jnp.float32),
        ],
    )
    def k(x_hbm, pe2_hbm, ident_hbm, table_hbm, out_hbm,
          idx_v, pe2_v, ident_v, rows_v, acc_s):
        sid = lax.axis_index("s")
        wid = sid * 2 + lax.axis_index("c")
        pltpu.sync_copy(x_hbm.at[wid], idx_v)
        pltpu.sync_copy(pe2_hbm, pe2_v)
        pltpu.sync_copy(ident_hbm, ident_v)
        acc = acc_s.at[sid]

        @pl.loop(0, NCHUNK)
        def _(c):
            pltpu.sync_copy(table_hbm.at[idx_v.at[c]], rows_v)
            phase = lax.rem(c * C, SEQ)
            pltpu.sync_copy(pe2_v.at[pl.ds(phase, C)], acc)
            pltpu.sync_copy(rows_v, acc.at[ident_v], add=True)
            row0 = wid * RPW + c * C
            pltpu.sync_copy(acc, out_hbm.at[pl.ds(row0, C)])

    return k(x3, pe2, ident, table)


def kernel(x, table):
    x3 = x.astype(jnp.int32).reshape(NW, NCHUNK, C)
    out = _sc_embed(x3, table, _PE2, _IDENT)
    return out.reshape(BATCH, SEQ, D)
